# baseline (speedup 1.0000x reference)
import functools

import jax
import jax.numpy as jnp
from jax import lax
from jax.experimental import pallas as pl
from jax.experimental.pallas import tpu as pltpu
from jax.experimental.pallas import tpu_sc as plsc

N = 2048
D = 256
E = 65536
HID = 256
EMB = 128
NOISE_DIM = 16
AUG = 2
NEG = 10
NORM = 0.1
AUG_GAE_W = 1e-05
SIA_LOSS_W = 1e-05

_NSC = 2
_NTILE = 16
_BLK_ROWS = N // 4
_SP_WORDS = _BLK_ROWS * N
_TRASH = N
_EPT = E // _NTILE
_CHUNK = 128
_NCHUNK = _EPT // _CHUNK
_ZCH = 8192
_ZSTRIDE = (_SP_WORDS + _TRASH) // _NTILE
_DSTRIDE = _SP_WORDS // _NTILE

_sc_mesh = functools.partial(
    plsc.VectorSubcoreMesh, core_axis_name="c", subcore_axis_name="s")


@functools.partial(
    pl.kernel,
    out_type=(jax.ShapeDtypeStruct((N * N,), jnp.float32),
              jax.ShapeDtypeStruct((N * N,), jnp.float32)),
    mesh=_sc_mesh(),
    scratch_types=[
        pltpu.VMEM_SHARED((_SP_WORDS + _TRASH,), jnp.float32),
        pltpu.VMEM((_EPT,), jnp.int32),
        pltpu.VMEM((_EPT,), jnp.int32),
        pltpu.VMEM((_EPT,), jnp.float32),
        pltpu.VMEM((_NCHUNK, _CHUNK), jnp.int32),
        pltpu.VMEM((_NCHUNK, _CHUNK), jnp.float32),
        pltpu.VMEM((_ZCH,), jnp.float32),
        pltpu.SemaphoreType.DMA,
    ],
)
def _build_dense(erow, ecol, ew, orow, ocol, a_out, l_out,
                 spm, rbuf, cbuf, wbuf, idxbuf, valbuf, zbuf, sem):
    c = lax.axis_index("c")
    s = lax.axis_index("s")

    zero16 = jnp.zeros((16,), jnp.float32)

    def _zfill(i, carry):
        zbuf[pl.ds(i * 16, 16)] = zero16
        return carry
    lax.fori_loop(0, _ZCH // 16, _zfill, 0)

    for p in range(4):
        is_a = p < 2
        base = (c * 2 + (p % 2)) * _BLK_ROWS
        out = a_out if is_a else l_out
        r_src = erow if is_a else orow
        c_src = ecol if is_a else ocol

        for k in range(_ZSTRIDE // _ZCH):
            pltpu.sync_copy(zbuf, spm.at[pl.ds(s * _ZSTRIDE + k * _ZCH, _ZCH)])
        rem = _ZSTRIDE % _ZCH
        if rem:
            pltpu.sync_copy(zbuf.at[pl.ds(0, rem)],
                            spm.at[pl.ds(s * _ZSTRIDE + _ZSTRIDE - rem, rem)])
        plsc.subcore_barrier()

        eb = s * _EPT
        pltpu.sync_copy(r_src.at[pl.ds(eb, _EPT)], rbuf)
        pltpu.sync_copy(c_src.at[pl.ds(eb, _EPT)], cbuf)
        if is_a:
            pltpu.sync_copy(ew.at[pl.ds(eb, _EPT)], wbuf)

        copies = []
        for j in range(_NCHUNK):
            def _grp(g, carry, _j=j):
                o = _j * _CHUNK + g * 16
                r16 = rbuf[pl.ds(o, 16)]
                c16 = cbuf[pl.ds(o, 16)]
                inb = (r16 >= base) & (r16 < base + _BLK_ROWS)
                idx16 = jnp.where(inb, (r16 - base) * N + c16,
                                  _SP_WORDS + c16)
                if is_a:
                    v16 = jnp.where(inb, wbuf[pl.ds(o, 16)], 0.0)
                else:
                    v16 = jnp.where(inb, 1.0, 0.0)
                idxbuf[_j, pl.ds(g * 16, 16)] = idx16
                valbuf[_j, pl.ds(g * 16, 16)] = v16
                return carry
            lax.fori_loop(0, _CHUNK // 16, _grp, 0)
            copies.append(
                pltpu.async_copy(valbuf.at[j], spm.at[idxbuf.at[j]], sem,
                                 add=True))
        for cp in copies:
            cp.wait()
        plsc.subcore_barrier()

        pltpu.sync_copy(spm.at[pl.ds(s * _DSTRIDE, _DSTRIDE)],
                        out.at[pl.ds(base * N + s * _DSTRIDE, _DSTRIDE)])
        plsc.subcore_barrier()


_GB = AUG * N * NEG
_GW = _GB // (_NSC * _NTILE)
_GCH = 128


@functools.partial(
    pl.kernel,
    out_type=jax.ShapeDtypeStruct((_GB, EMB), jnp.float32),
    mesh=_sc_mesh(),
    scratch_types=[
        pltpu.VMEM((_GW,), jnp.int32),
        pltpu.VMEM((_GCH, EMB), jnp.float32),
        pltpu.VMEM((_GCH, EMB), jnp.float32),
        pltpu.SemaphoreType.DMA,
        pltpu.SemaphoreType.DMA,
        pltpu.SemaphoreType.DMA,
        pltpu.SemaphoreType.DMA,
    ],
)
def _neg_gather(h_hbm, idx_hbm, out_hbm, idx_v, rows_a, rows_b,
                gsem_a, gsem_b, wsem_a, wsem_b):
    c = lax.axis_index("c")
    s = lax.axis_index("s")
    wid = s * _NSC + c
    base = wid * _GW
    pltpu.sync_copy(idx_hbm.at[pl.ds(base, _GW)], idx_v)

    bufs = ((rows_a, gsem_a, wsem_a), (rows_b, gsem_b, wsem_b))
    nch = _GW // _GCH
    gathers = [None, None]
    writes = [None, None]

    def _gather(g):
        buf, gsem, _ = bufs[g % 2]
        return pltpu.async_copy(h_hbm.at[idx_v.at[pl.ds(g * _GCH, _GCH)]],
                                buf, gsem)

    gathers[0] = _gather(0)
    for g in range(nch):
        b = g % 2
        nb = (g + 1) % 2
        gathers[b].wait()
        if g + 1 < nch:
            if writes[nb] is not None:
                writes[nb].wait()
            gathers[nb] = _gather(g + 1)
        buf, _, wsem = bufs[b]
        writes[b] = pltpu.async_copy(
            buf, out_hbm.at[pl.ds(base + g * _GCH, _GCH)], wsem)
    writes[0].wait()
    writes[1].wait()


def _matmul_body(x_ref, w_ref, out_ref):
    out_ref[...] = jnp.dot(x_ref[...], w_ref[...],
                           preferred_element_type=jnp.float32)


def _matmul(x, w):
    return pl.pallas_call(
        _matmul_body,
        out_shape=jax.ShapeDtypeStruct((x.shape[0], w.shape[1]), jnp.float32),
    )(x, w)


def _gcn_xw_body(a_ref, xw_ref, b_ref, out_ref):
    agg = jnp.dot(a_ref[...], xw_ref[...], preferred_element_type=jnp.float32)
    out_ref[...] = jnp.maximum(agg + b_ref[...], 0.0)


def _gcn_layer_pre(a, xw, b):
    hdim = xw.shape[1]
    blk = 256
    return pl.pallas_call(
        _gcn_xw_body,
        grid=(N // blk,),
        in_specs=[pl.BlockSpec((blk, N), lambda i: (i, 0)),
                  pl.BlockSpec((N, hdim), lambda i: (0, 0)),
                  pl.BlockSpec((1, hdim), lambda i: (0, 0))],
        out_specs=pl.BlockSpec((blk, hdim), lambda i: (i, 0)),
        out_shape=jax.ShapeDtypeStruct((N, hdim), jnp.float32),
    )(a, xw, b.reshape(1, hdim))


def _gcn_body(a_ref, xin_ref, w_ref, b_ref, out_ref, xw_ref):
    i = pl.program_id(0)

    @pl.when(i == 0)
    def _():
        xw_ref[...] = jnp.dot(xin_ref[...], w_ref[...],
                              preferred_element_type=jnp.float32)

    agg = jnp.dot(a_ref[...], xw_ref[...], preferred_element_type=jnp.float32)
    out_ref[...] = jnp.maximum(agg + b_ref[...], 0.0)


def _gcn_layer(a, xin, w, b):
    k, hdim = w.shape
    blk = 256
    return pl.pallas_call(
        _gcn_body,
        grid=(N // blk,),
        in_specs=[pl.BlockSpec((blk, N), lambda i: (i, 0)),
                  pl.BlockSpec((N, k), lambda i: (0, 0)),
                  pl.BlockSpec((k, hdim), lambda i: (0, 0)),
                  pl.BlockSpec((1, hdim), lambda i: (0, 0))],
        out_specs=pl.BlockSpec((blk, hdim), lambda i: (i, 0)),
        out_shape=jax.ShapeDtypeStruct((N, hdim), jnp.float32),
        scratch_shapes=[pltpu.VMEM((N, hdim), jnp.float32)],
    )(a, xin, w, b.reshape(1, hdim))


def _dao_pos_body(h_ref, nz_ref, dw1h_ref, dw1n_ref, db1_ref, a1_ref,
                  dw2_ref, db2_ref, a2_ref, swc_ref, aug_ref, cat_ref,
                  pos_ref):
    t = (jnp.dot(h_ref[...], dw1h_ref[...], preferred_element_type=jnp.float32)
         + jnp.dot(nz_ref[...], dw1n_ref[...],
                   preferred_element_type=jnp.float32)
         + db1_ref[...])
    t = jnp.maximum(t, 0.0) + a1_ref[...] * jnp.minimum(t, 0.0)
    u = jnp.dot(t, dw2_ref[...], preferred_element_type=jnp.float32) + db2_ref[...]
    aug = jnp.maximum(u, 0.0) + a2_ref[...] * jnp.minimum(u, 0.0)
    aug_ref[...] = aug
    cat_ref[0:N, :] = aug
    cat_ref[N:2 * N, :] = aug
    d = jnp.abs(aug - h_ref[...])
    logit = jnp.dot(d, swc_ref[...], preferred_element_type=jnp.float32)
    ce = jnp.log1p(jnp.exp(-jnp.abs(logit))) + jnp.maximum(-logit, 0.0)
    pos_ref[0, 0] = jnp.sum(ce)


def _dao_pos(h, noise, dw1, db1, a1, dw2, db2, a2, sw):
    return pl.pallas_call(
        _dao_pos_body,
        out_specs=(pl.BlockSpec(memory_space=pltpu.VMEM),
                   pl.BlockSpec(memory_space=pltpu.VMEM),
                   pl.BlockSpec(memory_space=pltpu.SMEM)),
        out_shape=(jax.ShapeDtypeStruct((N, EMB), jnp.float32),
                   jax.ShapeDtypeStruct((AUG * N, EMB), jnp.float32),
                   jax.ShapeDtypeStruct((1, 1), jnp.float32)),
    )(h, noise, dw1[:EMB], dw1[EMB:], db1.reshape(1, HID), a1.reshape(1, HID),
      dw2, db2.reshape(1, EMB), a2.reshape(1, EMB), sw.reshape(EMB, 1))


_CE_BI = 256
_CE_BJ = 1024


def _ce_body(hi_ref, hj_ref, ai_ref, lc_ref, s1_ref, s2_ref, acc_ref):
    i = pl.program_id(0)
    j = pl.program_id(1)

    @pl.when((i == 0) & (j == 0))
    def _():
        acc_ref[0] = 0.0
        acc_ref[1] = 0.0

    dn = (((1,), (1,)), ((), ()))
    rec1 = lax.dot_general(hi_ref[...], hj_ref[...], dn,
                           preferred_element_type=jnp.float32)
    rec2 = lax.dot_general(ai_ref[...], hj_ref[...], dn,
                           preferred_element_type=jnp.float32)
    one_m = 1.0 - (lc_ref[...] > 0.5).astype(jnp.float32)

    wce1 = one_m * rec1 + jnp.log1p(jnp.exp(-rec1))
    wce2 = (one_m * rec2 + jnp.log1p(jnp.exp(-jnp.abs(rec2)))
            + jnp.maximum(-rec2, 0.0))

    acc_ref[0] += jnp.sum(wce1)
    acc_ref[1] += jnp.sum(wce2)

    @pl.when((i == N // _CE_BI - 1) & (j == N // _CE_BJ - 1))
    def _():
        s1_ref[0, 0] = acc_ref[0]
        s2_ref[0, 0] = acc_ref[1]


def _ce_sums(h, aug_h, lc):
    return pl.pallas_call(
        _ce_body,
        grid=(N // _CE_BI, N // _CE_BJ),
        in_specs=[pl.BlockSpec((_CE_BI, EMB), lambda i, j: (i, 0)),
                  pl.BlockSpec((_CE_BJ, EMB), lambda i, j: (j, 0)),
                  pl.BlockSpec((_CE_BI, EMB), lambda i, j: (i, 0)),
                  pl.BlockSpec((_CE_BI, _CE_BJ), lambda i, j: (i, j))],
        out_specs=(pl.BlockSpec(memory_space=pltpu.SMEM),
                   pl.BlockSpec(memory_space=pltpu.SMEM)),
        out_shape=(jax.ShapeDtypeStruct((1, 1), jnp.float32),
                   jax.ShapeDtypeStruct((1, 1), jnp.float32)),
        scratch_shapes=[pltpu.SMEM((2,), jnp.float32)],
    )(h, h, aug_h, lc)


_SIA_B = 2048


def _sia_body(negb_ref, aug_ref, swc_ref, out_ref, acc_ref):
    j = pl.program_id(0)
    i = pl.program_id(1)

    @pl.when((j == 0) & (i == 0))
    def _():
        acc_ref[0] = 0.0

    d = jnp.abs(aug_ref[...] - negb_ref[...])
    logit = jnp.dot(d, swc_ref[...], preferred_element_type=jnp.float32)
    ce = logit + jnp.log1p(jnp.exp(-jnp.abs(logit))) + jnp.maximum(-logit, 0.0)
    acc_ref[0] += jnp.sum(ce)

    @pl.when((j == NEG - 1) & (i == AUG * N // _SIA_B - 1))
    def _():
        out_ref[0, 0] = acc_ref[0]


def _sia_neg_sum(neg_h, aug_h, sw):
    nblk = AUG * N // _SIA_B
    return pl.pallas_call(
        _sia_body,
        grid=(NEG, nblk),
        in_specs=[pl.BlockSpec((_SIA_B, EMB), lambda j, i: (j * nblk + i, 0)),
                  pl.BlockSpec((_SIA_B, EMB),
                               lambda j, i: (i % (N // _SIA_B), 0)),
                  pl.BlockSpec((EMB, 1), lambda j, i: (0, 0))],
        out_specs=pl.BlockSpec(memory_space=pltpu.SMEM),
        out_shape=jax.ShapeDtypeStruct((1, 1), jnp.float32),
        scratch_shapes=[pltpu.SMEM((1,), jnp.float32)],
    )(neg_h, aug_h, sw.reshape(EMB, 1))


def kernel(x, adj_weight, aug_noise, W1, b1, W2, b2, dW1, db1, a1, dW2, db2,
           a2, siamese_w, edge_index, adj_orig_index, negative_index):
    xw1 = _matmul(x, W1)
    a_flat, lc_flat = _build_dense(edge_index[0], edge_index[1], adj_weight,
                                   adj_orig_index[0], adj_orig_index[1])
    adj = a_flat.reshape(N, N)
    lc = lc_flat.reshape(N, N)
    h1 = _gcn_layer_pre(adj, xw1, b1)
    h = _gcn_layer(adj, h1, W2, b2)

    aug_h, aug_cat, pos_sum = _dao_pos(h, aug_noise, dW1, db1, a1, dW2, db2,
                                       a2, siamese_w)
    neg_h = _neg_gather(h, negative_index.T.reshape(-1))
    s1, s2 = _ce_sums(h, aug_h, lc)
    neg_sum = _sia_neg_sum(neg_h, aug_h, siamese_w)

    nn = float(N * N)
    gae_l = NORM * s1[0, 0] / nn
    aug_gae_l = (NORM * s2[0, 0] / nn) * AUG_GAE_W
    n_sia = float(AUG * N + AUG * N * NEG)
    sia_l = ((AUG * pos_sum[0, 0] + neg_sum[0, 0]) / n_sia) * SIA_LOSS_W
    total = gae_l + aug_gae_l + sia_l
    return total, gae_l, aug_gae_l, sia_l, h, aug_cat


# speedup vs baseline: 1.3194x; 1.3194x over previous
import functools

import jax
import jax.numpy as jnp
from jax import lax
from jax.experimental import pallas as pl
from jax.experimental.pallas import tpu as pltpu
from jax.experimental.pallas import tpu_sc as plsc

N = 2048
D = 256
E = 65536
HID = 256
EMB = 128
NOISE_DIM = 16
AUG = 2
NEG = 10
NORM = 0.1
AUG_GAE_W = 1e-05
SIA_LOSS_W = 1e-05

_NSC = 2
_NTILE = 16
_BLK_ROWS = N // 4
_SP_WORDS = _BLK_ROWS * N
_TRASH = N
_EPT = E // _NTILE
_CHUNK = 128
_NCHUNK = _EPT // _CHUNK
_ZCH = 8192
_ZSTRIDE = (_SP_WORDS + _TRASH) // _NTILE
_DSTRIDE = _SP_WORDS // _NTILE

_sc_mesh = functools.partial(
    plsc.VectorSubcoreMesh, core_axis_name="c", subcore_axis_name="s")


_BUILD_SCRATCH = [
    pltpu.VMEM_SHARED((_SP_WORDS + _TRASH,), jnp.float32),
    pltpu.VMEM((_EPT,), jnp.int32),
    pltpu.VMEM((_EPT,), jnp.int32),
    pltpu.VMEM((_EPT,), jnp.float32),
    pltpu.VMEM((_NCHUNK, _CHUNK), jnp.int32),
    pltpu.VMEM((_NCHUNK, _CHUNK), jnp.float32),
    pltpu.VMEM((_ZCH,), jnp.float32),
    pltpu.SemaphoreType.DMA,
]


def _scatter_build_body(use_w, erow, ecol, ew, out,
                        spm, rbuf, cbuf, wbuf, idxbuf, valbuf, zbuf, sem):
    c = lax.axis_index("c")
    s = lax.axis_index("s")

    zero16 = jnp.zeros((16,), jnp.float32)

    def _zfill(i, carry):
        zbuf[pl.ds(i * 16, 16)] = zero16
        return carry
    lax.fori_loop(0, _ZCH // 16, _zfill, 0)

    for p in range(2):
        base = (c * 2 + p) * _BLK_ROWS

        for k in range(_ZSTRIDE // _ZCH):
            pltpu.sync_copy(zbuf, spm.at[pl.ds(s * _ZSTRIDE + k * _ZCH, _ZCH)])
        rem = _ZSTRIDE % _ZCH
        if rem:
            pltpu.sync_copy(zbuf.at[pl.ds(0, rem)],
                            spm.at[pl.ds(s * _ZSTRIDE + _ZSTRIDE - rem, rem)])
        plsc.subcore_barrier()

        eb = s * _EPT
        pltpu.sync_copy(erow.at[pl.ds(eb, _EPT)], rbuf)
        pltpu.sync_copy(ecol.at[pl.ds(eb, _EPT)], cbuf)
        if use_w:
            pltpu.sync_copy(ew.at[pl.ds(eb, _EPT)], wbuf)

        copies = []
        for j in range(_NCHUNK):
            def _grp(g, carry, _j=j):
                o = _j * _CHUNK + g * 16
                r16 = rbuf[pl.ds(o, 16)]
                c16 = cbuf[pl.ds(o, 16)]
                inb = (r16 >= base) & (r16 < base + _BLK_ROWS)
                idx16 = jnp.where(inb, (r16 - base) * N + c16,
                                  _SP_WORDS + c16)
                if use_w:
                    v16 = jnp.where(inb, wbuf[pl.ds(o, 16)], 0.0)
                else:
                    v16 = jnp.where(inb, 1.0, 0.0)
                idxbuf[_j, pl.ds(g * 16, 16)] = idx16
                valbuf[_j, pl.ds(g * 16, 16)] = v16
                return carry
            lax.fori_loop(0, _CHUNK // 16, _grp, 0)
            copies.append(
                pltpu.async_copy(valbuf.at[j], spm.at[idxbuf.at[j]], sem,
                                 add=True))
        for cp in copies:
            cp.wait()
        plsc.subcore_barrier()

        pltpu.sync_copy(spm.at[pl.ds(s * _DSTRIDE, _DSTRIDE)],
                        out.at[pl.ds(base * N + s * _DSTRIDE, _DSTRIDE)])
        plsc.subcore_barrier()


@functools.partial(
    pl.kernel,
    out_type=jax.ShapeDtypeStruct((N * N,), jnp.float32),
    mesh=_sc_mesh(),
    scratch_types=_BUILD_SCRATCH,
)
def _build_adj(erow, ecol, ew, out, *scratch):
    _scatter_build_body(True, erow, ecol, ew, out, *scratch)


@functools.partial(
    pl.kernel,
    out_type=jax.ShapeDtypeStruct((N * N,), jnp.float32),
    mesh=_sc_mesh(),
    scratch_types=_BUILD_SCRATCH,
)
def _build_lab(erow, ecol, out, *scratch):
    _scatter_build_body(False, erow, ecol, None, out, *scratch)


_GB = AUG * N * NEG
_GW = _GB // (_NSC * _NTILE)
_GCH = 128


@functools.partial(
    pl.kernel,
    out_type=jax.ShapeDtypeStruct((_GB, EMB), jnp.float32),
    mesh=_sc_mesh(),
    scratch_types=[
        pltpu.VMEM((_GW,), jnp.int32),
        pltpu.VMEM((_GCH, EMB), jnp.float32),
        pltpu.VMEM((_GCH, EMB), jnp.float32),
        pltpu.SemaphoreType.DMA,
        pltpu.SemaphoreType.DMA,
        pltpu.SemaphoreType.DMA,
        pltpu.SemaphoreType.DMA,
    ],
)
def _neg_gather(h_hbm, idx_hbm, out_hbm, idx_v, rows_a, rows_b,
                gsem_a, gsem_b, wsem_a, wsem_b):
    c = lax.axis_index("c")
    s = lax.axis_index("s")
    wid = s * _NSC + c
    base = wid * _GW
    pltpu.sync_copy(idx_hbm.at[pl.ds(base, _GW)], idx_v)

    bufs = ((rows_a, gsem_a, wsem_a), (rows_b, gsem_b, wsem_b))
    nch = _GW // _GCH
    gathers = [None, None]
    writes = [None, None]

    def _gather(g):
        buf, gsem, _ = bufs[g % 2]
        return pltpu.async_copy(h_hbm.at[idx_v.at[pl.ds(g * _GCH, _GCH)]],
                                buf, gsem)

    gathers[0] = _gather(0)
    for g in range(nch):
        b = g % 2
        nb = (g + 1) % 2
        gathers[b].wait()
        if g + 1 < nch:
            if writes[nb] is not None:
                writes[nb].wait()
            gathers[nb] = _gather(g + 1)
        buf, _, wsem = bufs[b]
        writes[b] = pltpu.async_copy(
            buf, out_hbm.at[pl.ds(base + g * _GCH, _GCH)], wsem)
    writes[0].wait()
    writes[1].wait()


def _matmul_body(x_ref, w_ref, out_ref):
    out_ref[...] = jnp.dot(x_ref[...], w_ref[...],
                           preferred_element_type=jnp.float32)


def _matmul(x, w):
    return pl.pallas_call(
        _matmul_body,
        out_shape=jax.ShapeDtypeStruct((x.shape[0], w.shape[1]), jnp.float32),
    )(x, w)


_GCN_BLK = 256


def _gcn_xw_body(a_ref, xw_ref, b_ref, out_ref):
    a2 = a_ref[...].reshape(_GCN_BLK, N)
    agg = jnp.dot(a2, xw_ref[...], preferred_element_type=jnp.float32)
    out_ref[...] = jnp.maximum(agg + b_ref[...], 0.0)


def _gcn_layer_pre(a_flat, xw, b):
    hdim = xw.shape[1]
    return pl.pallas_call(
        _gcn_xw_body,
        grid=(N // _GCN_BLK,),
        in_specs=[pl.BlockSpec((_GCN_BLK * N,), lambda i: (i,)),
                  pl.BlockSpec((N, hdim), lambda i: (0, 0)),
                  pl.BlockSpec((1, hdim), lambda i: (0, 0))],
        out_specs=pl.BlockSpec((_GCN_BLK, hdim), lambda i: (i, 0)),
        out_shape=jax.ShapeDtypeStruct((N, hdim), jnp.float32),
    )(a_flat, xw, b.reshape(1, hdim))


def _gcn_body(a_ref, xin_ref, w_ref, b_ref, out_ref, xw_ref):
    i = pl.program_id(0)

    @pl.when(i == 0)
    def _():
        xw_ref[...] = jnp.dot(xin_ref[...], w_ref[...],
                              preferred_element_type=jnp.float32)

    a2 = a_ref[...].reshape(_GCN_BLK, N)
    agg = jnp.dot(a2, xw_ref[...], preferred_element_type=jnp.float32)
    out_ref[...] = jnp.maximum(agg + b_ref[...], 0.0)


def _gcn_layer(a_flat, xin, w, b):
    k, hdim = w.shape
    return pl.pallas_call(
        _gcn_body,
        grid=(N // _GCN_BLK,),
        in_specs=[pl.BlockSpec((_GCN_BLK * N,), lambda i: (i,)),
                  pl.BlockSpec((N, k), lambda i: (0, 0)),
                  pl.BlockSpec((k, hdim), lambda i: (0, 0)),
                  pl.BlockSpec((1, hdim), lambda i: (0, 0))],
        out_specs=pl.BlockSpec((_GCN_BLK, hdim), lambda i: (i, 0)),
        out_shape=jax.ShapeDtypeStruct((N, hdim), jnp.float32),
        scratch_shapes=[pltpu.VMEM((N, hdim), jnp.float32)],
    )(a_flat, xin, w, b.reshape(1, hdim))


def _dao_pos_body(h_ref, nz_ref, dw1h_ref, dw1n_ref, db1_ref, a1_ref,
                  dw2_ref, db2_ref, a2_ref, swc_ref, aug_ref, cat_ref,
                  pos_ref):
    t = (jnp.dot(h_ref[...], dw1h_ref[...], preferred_element_type=jnp.float32)
         + jnp.dot(nz_ref[...], dw1n_ref[...],
                   preferred_element_type=jnp.float32)
         + db1_ref[...])
    t = jnp.maximum(t, 0.0) + a1_ref[...] * jnp.minimum(t, 0.0)
    u = jnp.dot(t, dw2_ref[...], preferred_element_type=jnp.float32) + db2_ref[...]
    aug = jnp.maximum(u, 0.0) + a2_ref[...] * jnp.minimum(u, 0.0)
    aug_ref[...] = aug
    cat_ref[0:N, :] = aug
    cat_ref[N:2 * N, :] = aug
    d = jnp.abs(aug - h_ref[...])
    logit = jnp.dot(d, swc_ref[...], preferred_element_type=jnp.float32)
    ce = jnp.log1p(jnp.exp(-jnp.abs(logit))) + jnp.maximum(-logit, 0.0)
    pos_ref[0, 0] = jnp.sum(ce)


def _dao_pos(h, noise, dw1, db1, a1, dw2, db2, a2, sw):
    return pl.pallas_call(
        _dao_pos_body,
        out_specs=(pl.BlockSpec(memory_space=pltpu.VMEM),
                   pl.BlockSpec(memory_space=pltpu.VMEM),
                   pl.BlockSpec(memory_space=pltpu.SMEM)),
        out_shape=(jax.ShapeDtypeStruct((N, EMB), jnp.float32),
                   jax.ShapeDtypeStruct((AUG * N, EMB), jnp.float32),
                   jax.ShapeDtypeStruct((1, 1), jnp.float32)),
    )(h, noise, dw1[:EMB], dw1[EMB:], db1.reshape(1, HID), a1.reshape(1, HID),
      dw2, db2.reshape(1, EMB), a2.reshape(1, EMB), sw.reshape(EMB, 1))


_CE_BI = 256


def _ce_body(hi_ref, hj_ref, ai_ref, lc_ref, s1_ref, s2_ref, acc_ref):
    i = pl.program_id(0)

    @pl.when(i == 0)
    def _():
        acc_ref[0] = 0.0
        acc_ref[1] = 0.0

    dn = (((1,), (1,)), ((), ()))
    rec1 = lax.dot_general(hi_ref[...], hj_ref[...], dn,
                           preferred_element_type=jnp.float32)
    rec2 = lax.dot_general(ai_ref[...], hj_ref[...], dn,
                           preferred_element_type=jnp.float32)
    one_m = 1.0 - (lc_ref[...].reshape(_CE_BI, N) > 0.5).astype(jnp.float32)

    wce1 = one_m * rec1 + jnp.log1p(jnp.exp(-rec1))
    wce2 = (one_m * rec2 + jnp.log1p(jnp.exp(-jnp.abs(rec2)))
            + jnp.maximum(-rec2, 0.0))

    acc_ref[0] += jnp.sum(wce1)
    acc_ref[1] += jnp.sum(wce2)

    @pl.when(i == N // _CE_BI - 1)
    def _():
        s1_ref[0, 0] = acc_ref[0]
        s2_ref[0, 0] = acc_ref[1]


def _ce_sums(h, aug_h, lc_flat):
    return pl.pallas_call(
        _ce_body,
        grid=(N // _CE_BI,),
        in_specs=[pl.BlockSpec((_CE_BI, EMB), lambda i: (i, 0)),
                  pl.BlockSpec((N, EMB), lambda i: (0, 0)),
                  pl.BlockSpec((_CE_BI, EMB), lambda i: (i, 0)),
                  pl.BlockSpec((_CE_BI * N,), lambda i: (i,))],
        out_specs=(pl.BlockSpec(memory_space=pltpu.SMEM),
                   pl.BlockSpec(memory_space=pltpu.SMEM)),
        out_shape=(jax.ShapeDtypeStruct((1, 1), jnp.float32),
                   jax.ShapeDtypeStruct((1, 1), jnp.float32)),
        scratch_shapes=[pltpu.SMEM((2,), jnp.float32)],
    )(h, h, aug_h, lc_flat)


_SIA_B = 2048


def _sia_body(negb_ref, aug_ref, swc_ref, out_ref, acc_ref):
    j = pl.program_id(0)
    i = pl.program_id(1)

    @pl.when((j == 0) & (i == 0))
    def _():
        acc_ref[0] = 0.0

    d = jnp.abs(aug_ref[...] - negb_ref[...])
    logit = jnp.dot(d, swc_ref[...], preferred_element_type=jnp.float32)
    ce = logit + jnp.log1p(jnp.exp(-jnp.abs(logit))) + jnp.maximum(-logit, 0.0)
    acc_ref[0] += jnp.sum(ce)

    @pl.when((j == NEG - 1) & (i == AUG * N // _SIA_B - 1))
    def _():
        out_ref[0, 0] = acc_ref[0]


def _sia_neg_sum(neg_h, aug_h, sw):
    nblk = AUG * N // _SIA_B
    return pl.pallas_call(
        _sia_body,
        grid=(NEG, nblk),
        in_specs=[pl.BlockSpec((_SIA_B, EMB), lambda j, i: (j * nblk + i, 0)),
                  pl.BlockSpec((_SIA_B, EMB),
                               lambda j, i: (i % (N // _SIA_B), 0)),
                  pl.BlockSpec((EMB, 1), lambda j, i: (0, 0))],
        out_specs=pl.BlockSpec(memory_space=pltpu.SMEM),
        out_shape=jax.ShapeDtypeStruct((1, 1), jnp.float32),
        scratch_shapes=[pltpu.SMEM((1,), jnp.float32)],
    )(neg_h, aug_h, sw.reshape(EMB, 1))


def kernel(x, adj_weight, aug_noise, W1, b1, W2, b2, dW1, db1, a1, dW2, db2,
           a2, siamese_w, edge_index, adj_orig_index, negative_index):
    xw1 = _matmul(x, W1)
    a_flat = _build_adj(edge_index[0], edge_index[1], adj_weight)
    h1 = _gcn_layer_pre(a_flat, xw1, b1)
    lc_flat = _build_lab(adj_orig_index[0], adj_orig_index[1])
    h = _gcn_layer(a_flat, h1, W2, b2)

    aug_h, aug_cat, pos_sum = _dao_pos(h, aug_noise, dW1, db1, a1, dW2, db2,
                                       a2, siamese_w)
    neg_h = _neg_gather(h, negative_index.T.reshape(-1))
    s1, s2 = _ce_sums(h, aug_h, lc_flat)
    neg_sum = _sia_neg_sum(neg_h, aug_h, siamese_w)

    nn = float(N * N)
    gae_l = NORM * s1[0, 0] / nn
    aug_gae_l = (NORM * s2[0, 0] / nn) * AUG_GAE_W
    n_sia = float(AUG * N + AUG * N * NEG)
    sia_l = ((AUG * pos_sum[0, 0] + neg_sum[0, 0]) / n_sia) * SIA_LOSS_W
    total = gae_l + aug_gae_l + sia_l
    return total, gae_l, aug_gae_l, sia_l, h, aug_cat


# speedup vs baseline: 1.3654x; 1.0349x over previous
import functools

import jax
import jax.numpy as jnp
from jax import lax
from jax.experimental import pallas as pl
from jax.experimental.pallas import tpu as pltpu
from jax.experimental.pallas import tpu_sc as plsc

N = 2048
D = 256
E = 65536
HID = 256
EMB = 128
NOISE_DIM = 16
AUG = 2
NEG = 10
NORM = 0.1
AUG_GAE_W = 1e-05
SIA_LOSS_W = 1e-05

_NSC = 2
_NTILE = 16
_BLK_ROWS = N // 4
_SP_WORDS = _BLK_ROWS * N
_TRASH = N
_EPT = E // _NTILE
_CHUNK = 128
_NCHUNK = _EPT // _CHUNK
_ZCH = 8192
_ZSTRIDE = (_SP_WORDS + _TRASH) // _NTILE
_DSTRIDE = _SP_WORDS // _NTILE

_sc_mesh = functools.partial(
    plsc.VectorSubcoreMesh, core_axis_name="c", subcore_axis_name="s")


_BUILD_SCRATCH = [
    pltpu.VMEM_SHARED((_SP_WORDS + _TRASH,), jnp.float32),
    pltpu.VMEM((_EPT,), jnp.int32),
    pltpu.VMEM((_EPT,), jnp.int32),
    pltpu.VMEM((_EPT,), jnp.float32),
    pltpu.VMEM((_NCHUNK, _CHUNK), jnp.int32),
    pltpu.VMEM((_NCHUNK, _CHUNK), jnp.float32),
    pltpu.VMEM((_ZCH,), jnp.float32),
    pltpu.SemaphoreType.DMA,
]


def _scatter_build_body(use_w, erow, ecol, ew, out,
                        spm, rbuf, cbuf, wbuf, idxbuf, valbuf, zbuf, sem):
    c = lax.axis_index("c")
    s = lax.axis_index("s")

    zero16 = jnp.zeros((16,), jnp.float32)

    def _zfill(i, carry):
        zbuf[pl.ds(i * 16, 16)] = zero16
        return carry
    lax.fori_loop(0, _ZCH // 16, _zfill, 0)

    for p in range(2):
        base = (c * 2 + p) * _BLK_ROWS

        for k in range(_ZSTRIDE // _ZCH):
            pltpu.sync_copy(zbuf, spm.at[pl.ds(s * _ZSTRIDE + k * _ZCH, _ZCH)])
        rem = _ZSTRIDE % _ZCH
        if rem:
            pltpu.sync_copy(zbuf.at[pl.ds(0, rem)],
                            spm.at[pl.ds(s * _ZSTRIDE + _ZSTRIDE - rem, rem)])
        plsc.subcore_barrier()

        eb = s * _EPT
        pltpu.sync_copy(erow.at[pl.ds(eb, _EPT)], rbuf)
        pltpu.sync_copy(ecol.at[pl.ds(eb, _EPT)], cbuf)
        if use_w:
            pltpu.sync_copy(ew.at[pl.ds(eb, _EPT)], wbuf)

        copies = []
        for j in range(_NCHUNK):
            def _grp(g, carry, _j=j):
                o = _j * _CHUNK + g * 16
                r16 = rbuf[pl.ds(o, 16)]
                c16 = cbuf[pl.ds(o, 16)]
                inb = (r16 >= base) & (r16 < base + _BLK_ROWS)
                idx16 = jnp.where(inb, (r16 - base) * N + c16,
                                  _SP_WORDS + c16)
                if use_w:
                    v16 = jnp.where(inb, wbuf[pl.ds(o, 16)], 0.0)
                else:
                    v16 = jnp.where(inb, 1.0, 0.0)
                idxbuf[_j, pl.ds(g * 16, 16)] = idx16
                valbuf[_j, pl.ds(g * 16, 16)] = v16
                return carry
            lax.fori_loop(0, _CHUNK // 16, _grp, 0)
            copies.append(
                pltpu.async_copy(valbuf.at[j], spm.at[idxbuf.at[j]], sem,
                                 add=use_w))
        for cp in copies:
            cp.wait()
        plsc.subcore_barrier()

        pltpu.sync_copy(spm.at[pl.ds(s * _DSTRIDE, _DSTRIDE)],
                        out.at[pl.ds(base * N + s * _DSTRIDE, _DSTRIDE)])
        plsc.subcore_barrier()


@functools.partial(
    pl.kernel,
    out_type=jax.ShapeDtypeStruct((N * N,), jnp.float32),
    mesh=_sc_mesh(),
    scratch_types=_BUILD_SCRATCH,
)
def _build_adj(erow, ecol, ew, out, *scratch):
    _scatter_build_body(True, erow, ecol, ew, out, *scratch)


@functools.partial(
    pl.kernel,
    out_type=jax.ShapeDtypeStruct((N * N,), jnp.float32),
    mesh=_sc_mesh(),
    scratch_types=_BUILD_SCRATCH,
)
def _build_lab(erow, ecol, out, *scratch):
    _scatter_build_body(False, erow, ecol, None, out, *scratch)


_GB = AUG * N * NEG
_GW = _GB // (_NSC * _NTILE)
_GCH = 128


@functools.partial(
    pl.kernel,
    out_type=jax.ShapeDtypeStruct((_GB, EMB), jnp.float32),
    mesh=_sc_mesh(),
    scratch_types=[
        pltpu.VMEM((_GW,), jnp.int32),
        pltpu.VMEM((_GCH, EMB), jnp.float32),
        pltpu.VMEM((_GCH, EMB), jnp.float32),
        pltpu.SemaphoreType.DMA,
        pltpu.SemaphoreType.DMA,
        pltpu.SemaphoreType.DMA,
        pltpu.SemaphoreType.DMA,
    ],
)
def _neg_gather(h_hbm, idx_hbm, out_hbm, idx_v, rows_a, rows_b,
                gsem_a, gsem_b, wsem_a, wsem_b):
    c = lax.axis_index("c")
    s = lax.axis_index("s")
    wid = s * _NSC + c
    base = wid * _GW
    pltpu.sync_copy(idx_hbm.at[pl.ds(base, _GW)], idx_v)

    bufs = ((rows_a, gsem_a, wsem_a), (rows_b, gsem_b, wsem_b))
    nch = _GW // _GCH
    gathers = [None, None]
    writes = [None, None]

    def _gather(g):
        buf, gsem, _ = bufs[g % 2]
        return pltpu.async_copy(h_hbm.at[idx_v.at[pl.ds(g * _GCH, _GCH)]],
                                buf, gsem)

    gathers[0] = _gather(0)
    for g in range(nch):
        b = g % 2
        nb = (g + 1) % 2
        gathers[b].wait()
        if g + 1 < nch:
            if writes[nb] is not None:
                writes[nb].wait()
            gathers[nb] = _gather(g + 1)
        buf, _, wsem = bufs[b]
        writes[b] = pltpu.async_copy(
            buf, out_hbm.at[pl.ds(base + g * _GCH, _GCH)], wsem)
    writes[0].wait()
    writes[1].wait()


def _matmul_body(x_ref, w_ref, out_ref):
    out_ref[...] = jnp.dot(x_ref[...], w_ref[...],
                           preferred_element_type=jnp.float32)


def _matmul(x, w):
    return pl.pallas_call(
        _matmul_body,
        out_shape=jax.ShapeDtypeStruct((x.shape[0], w.shape[1]), jnp.float32),
    )(x, w)


_GCN_BLK = 256


def _gcn_xw_body(a_ref, xw_ref, b_ref, out_ref):
    a2 = a_ref[...].reshape(_GCN_BLK, N)
    agg = jnp.dot(a2, xw_ref[...], preferred_element_type=jnp.float32)
    out_ref[...] = jnp.maximum(agg + b_ref[...], 0.0)


def _gcn_layer_pre(a_flat, xw, b):
    hdim = xw.shape[1]
    return pl.pallas_call(
        _gcn_xw_body,
        grid=(N // _GCN_BLK,),
        in_specs=[pl.BlockSpec((_GCN_BLK * N,), lambda i: (i,)),
                  pl.BlockSpec((N, hdim), lambda i: (0, 0)),
                  pl.BlockSpec((1, hdim), lambda i: (0, 0))],
        out_specs=pl.BlockSpec((_GCN_BLK, hdim), lambda i: (i, 0)),
        out_shape=jax.ShapeDtypeStruct((N, hdim), jnp.float32),
    )(a_flat, xw, b.reshape(1, hdim))


def _gcn_body(a_ref, xin_ref, w_ref, b_ref, out_ref, xw_ref):
    i = pl.program_id(0)

    @pl.when(i == 0)
    def _():
        xw_ref[...] = jnp.dot(xin_ref[...], w_ref[...],
                              preferred_element_type=jnp.float32)

    a2 = a_ref[...].reshape(_GCN_BLK, N)
    agg = jnp.dot(a2, xw_ref[...], preferred_element_type=jnp.float32)
    out_ref[...] = jnp.maximum(agg + b_ref[...], 0.0)


def _gcn_layer(a_flat, xin, w, b):
    k, hdim = w.shape
    return pl.pallas_call(
        _gcn_body,
        grid=(N // _GCN_BLK,),
        in_specs=[pl.BlockSpec((_GCN_BLK * N,), lambda i: (i,)),
                  pl.BlockSpec((N, k), lambda i: (0, 0)),
                  pl.BlockSpec((k, hdim), lambda i: (0, 0)),
                  pl.BlockSpec((1, hdim), lambda i: (0, 0))],
        out_specs=pl.BlockSpec((_GCN_BLK, hdim), lambda i: (i, 0)),
        out_shape=jax.ShapeDtypeStruct((N, hdim), jnp.float32),
        scratch_shapes=[pltpu.VMEM((N, hdim), jnp.float32)],
    )(a_flat, xin, w, b.reshape(1, hdim))


def _dao_pos_body(h_ref, nz_ref, dw1h_ref, dw1n_ref, db1_ref, a1_ref,
                  dw2_ref, db2_ref, a2_ref, swc_ref, aug_ref, cat_ref,
                  pos_ref):
    t = (jnp.dot(h_ref[...], dw1h_ref[...], preferred_element_type=jnp.float32)
         + jnp.dot(nz_ref[...], dw1n_ref[...],
                   preferred_element_type=jnp.float32)
         + db1_ref[...])
    t = jnp.maximum(t, 0.0) + a1_ref[...] * jnp.minimum(t, 0.0)
    u = jnp.dot(t, dw2_ref[...], preferred_element_type=jnp.float32) + db2_ref[...]
    aug = jnp.maximum(u, 0.0) + a2_ref[...] * jnp.minimum(u, 0.0)
    aug_ref[...] = aug
    cat_ref[0:N, :] = aug
    cat_ref[N:2 * N, :] = aug
    d = jnp.abs(aug - h_ref[...])
    logit = jnp.dot(d, swc_ref[...], preferred_element_type=jnp.float32)
    ce = jnp.log1p(jnp.exp(-jnp.abs(logit))) + jnp.maximum(-logit, 0.0)
    pos_ref[0, 0] = jnp.sum(ce)
    cs_h = jnp.sum(h_ref[...], axis=0, keepdims=True)
    cs_a = jnp.sum(aug, axis=0, keepdims=True)
    pos_ref[0, 1] = jnp.sum(cs_h * cs_h)
    pos_ref[0, 2] = jnp.sum(cs_a * cs_h)


def _dao_pos(h, noise, dw1, db1, a1, dw2, db2, a2, sw):
    return pl.pallas_call(
        _dao_pos_body,
        out_specs=(pl.BlockSpec(memory_space=pltpu.VMEM),
                   pl.BlockSpec(memory_space=pltpu.VMEM),
                   pl.BlockSpec(memory_space=pltpu.SMEM)),
        out_shape=(jax.ShapeDtypeStruct((N, EMB), jnp.float32),
                   jax.ShapeDtypeStruct((AUG * N, EMB), jnp.float32),
                   jax.ShapeDtypeStruct((1, 3), jnp.float32)),
    )(h, noise, dw1[:EMB], dw1[EMB:], db1.reshape(1, HID), a1.reshape(1, HID),
      dw2, db2.reshape(1, EMB), a2.reshape(1, EMB), sw.reshape(EMB, 1))


def _labmm_body(lab_ref, h_ref, aug_ref, hi_ref, out_ref, acc_ref):
    i = pl.program_id(0)

    @pl.when(i == 0)
    def _():
        acc_ref[0] = 0.0
        acc_ref[1] = 0.0

    lab2 = lab_ref[...].reshape(_GCN_BLK, N)
    m = jnp.dot(lab2, h_ref[...], preferred_element_type=jnp.float32)
    acc_ref[0] += jnp.sum(hi_ref[...] * m)
    acc_ref[1] += jnp.sum(aug_ref[...] * m)

    @pl.when(i == N // _GCN_BLK - 1)
    def _():
        out_ref[0, 0] = acc_ref[0]
        out_ref[0, 1] = acc_ref[1]


def _labmm_sums(lab_flat, h, aug_h):
    return pl.pallas_call(
        _labmm_body,
        grid=(N // _GCN_BLK,),
        in_specs=[pl.BlockSpec((_GCN_BLK * N,), lambda i: (i,)),
                  pl.BlockSpec((N, EMB), lambda i: (0, 0)),
                  pl.BlockSpec((_GCN_BLK, EMB), lambda i: (i, 0)),
                  pl.BlockSpec((_GCN_BLK, EMB), lambda i: (i, 0))],
        out_specs=pl.BlockSpec(memory_space=pltpu.SMEM),
        out_shape=jax.ShapeDtypeStruct((1, 2), jnp.float32),
        scratch_shapes=[pltpu.SMEM((2,), jnp.float32)],
    )(lab_flat, h, aug_h, h)


_CE_BI = 256


def _ce_body(hi_ref, hj_ref, ai_ref, s_ref, acc_ref):
    i = pl.program_id(0)

    @pl.when(i == 0)
    def _():
        acc_ref[0] = 0.0
        acc_ref[1] = 0.0

    dn = (((1,), (1,)), ((), ()))
    rec1 = lax.dot_general(hi_ref[...], hj_ref[...], dn,
                           preferred_element_type=jnp.float32)
    rec2 = lax.dot_general(ai_ref[...], hj_ref[...], dn,
                           preferred_element_type=jnp.float32)

    sp1 = jnp.log1p(jnp.exp(-rec1))
    sp2 = jnp.log1p(jnp.exp(-jnp.abs(rec2))) + jnp.maximum(-rec2, 0.0)

    acc_ref[0] += jnp.sum(sp1)
    acc_ref[1] += jnp.sum(sp2)

    @pl.when(i == N // _CE_BI - 1)
    def _():
        s_ref[0, 0] = acc_ref[0]
        s_ref[0, 1] = acc_ref[1]


def _ce_sums(h, aug_h):
    return pl.pallas_call(
        _ce_body,
        grid=(N // _CE_BI,),
        in_specs=[pl.BlockSpec((_CE_BI, EMB), lambda i: (i, 0)),
                  pl.BlockSpec((N, EMB), lambda i: (0, 0)),
                  pl.BlockSpec((_CE_BI, EMB), lambda i: (i, 0))],
        out_specs=pl.BlockSpec(memory_space=pltpu.SMEM),
        out_shape=jax.ShapeDtypeStruct((1, 2), jnp.float32),
        scratch_shapes=[pltpu.SMEM((2,), jnp.float32)],
    )(h, h, aug_h)


_SIA_B = 2048


def _sia_body(negb_ref, aug_ref, swc_ref, out_ref, acc_ref):
    j = pl.program_id(0)
    i = pl.program_id(1)

    @pl.when((j == 0) & (i == 0))
    def _():
        acc_ref[0] = 0.0

    d = jnp.abs(aug_ref[...] - negb_ref[...])
    logit = jnp.dot(d, swc_ref[...], preferred_element_type=jnp.float32)
    ce = logit + jnp.log1p(jnp.exp(-jnp.abs(logit))) + jnp.maximum(-logit, 0.0)
    acc_ref[0] += jnp.sum(ce)

    @pl.when((j == NEG - 1) & (i == AUG * N // _SIA_B - 1))
    def _():
        out_ref[0, 0] = acc_ref[0]


def _sia_neg_sum(neg_h, aug_h, sw):
    nblk = AUG * N // _SIA_B
    return pl.pallas_call(
        _sia_body,
        grid=(NEG, nblk),
        in_specs=[pl.BlockSpec((_SIA_B, EMB), lambda j, i: (j * nblk + i, 0)),
                  pl.BlockSpec((_SIA_B, EMB),
                               lambda j, i: (i % (N // _SIA_B), 0)),
                  pl.BlockSpec((EMB, 1), lambda j, i: (0, 0))],
        out_specs=pl.BlockSpec(memory_space=pltpu.SMEM),
        out_shape=jax.ShapeDtypeStruct((1, 1), jnp.float32),
        scratch_shapes=[pltpu.SMEM((1,), jnp.float32)],
    )(neg_h, aug_h, sw.reshape(EMB, 1))


def kernel(x, adj_weight, aug_noise, W1, b1, W2, b2, dW1, db1, a1, dW2, db2,
           a2, siamese_w, edge_index, adj_orig_index, negative_index):
    xw1 = _matmul(x, W1)
    a_flat = _build_adj(edge_index[0], edge_index[1], adj_weight)
    h1 = _gcn_layer_pre(a_flat, xw1, b1)
    lc_flat = _build_lab(adj_orig_index[0], adj_orig_index[1])
    h = _gcn_layer(a_flat, h1, W2, b2)

    aug_h, aug_cat, dao_sums = _dao_pos(h, aug_noise, dW1, db1, a1, dW2, db2,
                                        a2, siamese_w)
    neg_h = _neg_gather(h, negative_index.T.reshape(-1))
    sp_sums = _ce_sums(h, aug_h)
    lab_sums = _labmm_sums(lc_flat, h, aug_h)
    neg_sum = _sia_neg_sum(neg_h, aug_h, siamese_w)

    pos_sum = dao_sums[0, 0]
    s1 = dao_sums[0, 1] - lab_sums[0, 0] + sp_sums[0, 0]
    s2 = dao_sums[0, 2] - lab_sums[0, 1] + sp_sums[0, 1]

    nn = float(N * N)
    gae_l = NORM * s1 / nn
    aug_gae_l = (NORM * s2 / nn) * AUG_GAE_W
    n_sia = float(AUG * N + AUG * N * NEG)
    sia_l = ((AUG * pos_sum + neg_sum[0, 0]) / n_sia) * SIA_LOSS_W
    total = gae_l + aug_gae_l + sia_l
    return total, gae_l, aug_gae_l, sia_l, h, aug_cat


# speedup vs baseline: 1.4074x; 1.0307x over previous
import functools

import jax
import jax.numpy as jnp
from jax import lax
from jax.experimental import pallas as pl
from jax.experimental.pallas import tpu as pltpu
from jax.experimental.pallas import tpu_sc as plsc

N = 2048
D = 256
E = 65536
HID = 256
EMB = 128
NOISE_DIM = 16
AUG = 2
NEG = 10
NORM = 0.1
AUG_GAE_W = 1e-05
SIA_LOSS_W = 1e-05

_NSC = 2
_NTILE = 16
_BLK_ROWS = N // 4
_SP_WORDS = _BLK_ROWS * N
_TRASH = N
_EPT = E // _NTILE
_CHUNK = 128
_NCHUNK = _EPT // _CHUNK
_ZCH = 8192
_ZSTRIDE = (_SP_WORDS + _TRASH) // _NTILE
_DSTRIDE = _SP_WORDS // _NTILE

_sc_mesh = functools.partial(
    plsc.VectorSubcoreMesh, core_axis_name="c", subcore_axis_name="s")


_BUILD_SCRATCH = [
    pltpu.VMEM_SHARED((_SP_WORDS + _TRASH,), jnp.float32),
    pltpu.VMEM((_EPT,), jnp.int32),
    pltpu.VMEM((_EPT,), jnp.int32),
    pltpu.VMEM((_EPT,), jnp.float32),
    pltpu.VMEM((_NCHUNK, _CHUNK), jnp.int32),
    pltpu.VMEM((_NCHUNK, _CHUNK), jnp.float32),
    pltpu.VMEM((_ZCH,), jnp.float32),
    pltpu.SemaphoreType.DMA,
]


def _scatter_build_body(use_w, erow, ecol, ew, out,
                        spm, rbuf, cbuf, wbuf, idxbuf, valbuf, zbuf, sem):
    c = lax.axis_index("c")
    s = lax.axis_index("s")

    zero16 = jnp.zeros((16,), jnp.float32)

    def _zfill(i, carry):
        zbuf[pl.ds(i * 16, 16)] = zero16
        return carry
    lax.fori_loop(0, _ZCH // 16, _zfill, 0)

    for p in range(2):
        base = (c * 2 + p) * _BLK_ROWS

        for k in range(_ZSTRIDE // _ZCH):
            pltpu.sync_copy(zbuf, spm.at[pl.ds(s * _ZSTRIDE + k * _ZCH, _ZCH)])
        rem = _ZSTRIDE % _ZCH
        if rem:
            pltpu.sync_copy(zbuf.at[pl.ds(0, rem)],
                            spm.at[pl.ds(s * _ZSTRIDE + _ZSTRIDE - rem, rem)])
        plsc.subcore_barrier()

        eb = s * _EPT
        pltpu.sync_copy(erow.at[pl.ds(eb, _EPT)], rbuf)
        pltpu.sync_copy(ecol.at[pl.ds(eb, _EPT)], cbuf)
        if use_w:
            pltpu.sync_copy(ew.at[pl.ds(eb, _EPT)], wbuf)

        copies = []
        for j in range(_NCHUNK):
            def _grp(g, carry, _j=j):
                o = _j * _CHUNK + g * 16
                r16 = rbuf[pl.ds(o, 16)]
                c16 = cbuf[pl.ds(o, 16)]
                inb = (r16 >= base) & (r16 < base + _BLK_ROWS)
                idx16 = jnp.where(inb, (r16 - base) * N + c16,
                                  _SP_WORDS + c16)
                if use_w:
                    v16 = jnp.where(inb, wbuf[pl.ds(o, 16)], 0.0)
                else:
                    v16 = jnp.where(inb, 1.0, 0.0)
                idxbuf[_j, pl.ds(g * 16, 16)] = idx16
                valbuf[_j, pl.ds(g * 16, 16)] = v16
                return carry
            lax.fori_loop(0, _CHUNK // 16, _grp, 0)
            copies.append(
                pltpu.async_copy(valbuf.at[j], spm.at[idxbuf.at[j]], sem,
                                 add=use_w))
        for cp in copies:
            cp.wait()
        plsc.subcore_barrier()

        pltpu.sync_copy(spm.at[pl.ds(s * _DSTRIDE, _DSTRIDE)],
                        out.at[pl.ds(base * N + s * _DSTRIDE, _DSTRIDE)])
        plsc.subcore_barrier()


@functools.partial(
    pl.kernel,
    out_type=jax.ShapeDtypeStruct((N * N,), jnp.float32),
    mesh=_sc_mesh(),
    scratch_types=_BUILD_SCRATCH,
)
def _build_adj(erow, ecol, ew, out, *scratch):
    _scatter_build_body(True, erow, ecol, ew, out, *scratch)


@functools.partial(
    pl.kernel,
    out_type=jax.ShapeDtypeStruct((N * N,), jnp.float32),
    mesh=_sc_mesh(),
    scratch_types=_BUILD_SCRATCH,
)
def _build_lab(erow, ecol, out, *scratch):
    _scatter_build_body(False, erow, ecol, None, out, *scratch)


_GB = AUG * N * NEG
_GW = _GB // (_NSC * _NTILE)
_GCH = 128


@functools.partial(
    pl.kernel,
    out_type=jax.ShapeDtypeStruct((_GB, EMB), jnp.float32),
    mesh=_sc_mesh(),
    scratch_types=[
        pltpu.VMEM((_GW,), jnp.int32),
        pltpu.VMEM((_GCH, EMB), jnp.float32),
        pltpu.VMEM((_GCH, EMB), jnp.float32),
        pltpu.SemaphoreType.DMA,
        pltpu.SemaphoreType.DMA,
        pltpu.SemaphoreType.DMA,
        pltpu.SemaphoreType.DMA,
    ],
)
def _neg_gather(h_hbm, idx_hbm, out_hbm, idx_v, rows_a, rows_b,
                gsem_a, gsem_b, wsem_a, wsem_b):
    c = lax.axis_index("c")
    s = lax.axis_index("s")
    wid = s * _NSC + c
    base = wid * _GW
    pltpu.sync_copy(idx_hbm.at[pl.ds(base, _GW)], idx_v)

    bufs = ((rows_a, gsem_a, wsem_a), (rows_b, gsem_b, wsem_b))
    nch = _GW // _GCH
    gathers = [None, None]
    writes = [None, None]

    def _gather(g):
        buf, gsem, _ = bufs[g % 2]
        return pltpu.async_copy(h_hbm.at[idx_v.at[pl.ds(g * _GCH, _GCH)]],
                                buf, gsem)

    gathers[0] = _gather(0)
    for g in range(nch):
        b = g % 2
        nb = (g + 1) % 2
        gathers[b].wait()
        if g + 1 < nch:
            if writes[nb] is not None:
                writes[nb].wait()
            gathers[nb] = _gather(g + 1)
        buf, _, wsem = bufs[b]
        writes[b] = pltpu.async_copy(
            buf, out_hbm.at[pl.ds(base + g * _GCH, _GCH)], wsem)
    writes[0].wait()
    writes[1].wait()


def _matmul_body(x_ref, w_ref, out_ref):
    out_ref[...] = jnp.dot(x_ref[...], w_ref[...],
                           preferred_element_type=jnp.float32)


def _matmul(x, w):
    return pl.pallas_call(
        _matmul_body,
        out_shape=jax.ShapeDtypeStruct((x.shape[0], w.shape[1]), jnp.float32),
    )(x, w)


_GCN_BLK = 256


def _gcn_xw_body(a_ref, xw_ref, b_ref, out_ref):
    a2 = a_ref[...].reshape(_GCN_BLK, N)
    agg = jnp.dot(a2, xw_ref[...], preferred_element_type=jnp.float32)
    out_ref[...] = jnp.maximum(agg + b_ref[...], 0.0)


def _gcn_layer_pre(a_flat, xw, b):
    hdim = xw.shape[1]
    return pl.pallas_call(
        _gcn_xw_body,
        grid=(N // _GCN_BLK,),
        in_specs=[pl.BlockSpec((_GCN_BLK * N,), lambda i: (i,)),
                  pl.BlockSpec((N, hdim), lambda i: (0, 0)),
                  pl.BlockSpec((1, hdim), lambda i: (0, 0))],
        out_specs=pl.BlockSpec((_GCN_BLK, hdim), lambda i: (i, 0)),
        out_shape=jax.ShapeDtypeStruct((N, hdim), jnp.float32),
    )(a_flat, xw, b.reshape(1, hdim))


def _gcn1_body(a_ref, xw1_ref, b_ref, w2_ref, xw2_ref, h1s_ref):
    i = pl.program_id(0)
    a2 = a_ref[...].reshape(_GCN_BLK, N)
    agg = jnp.dot(a2, xw1_ref[...], preferred_element_type=jnp.float32)
    h1s_ref[pl.ds(i * _GCN_BLK, _GCN_BLK), :] = jnp.maximum(
        agg + b_ref[...], 0.0)

    @pl.when(i == N // _GCN_BLK - 1)
    def _():
        xw2_ref[...] = jnp.dot(h1s_ref[...], w2_ref[...],
                               preferred_element_type=jnp.float32)


def _gcn1_fused(a_flat, xw1, b, w2):
    hdim = xw1.shape[1]
    return pl.pallas_call(
        _gcn1_body,
        grid=(N // _GCN_BLK,),
        in_specs=[pl.BlockSpec((_GCN_BLK * N,), lambda i: (i,)),
                  pl.BlockSpec((N, hdim), lambda i: (0, 0)),
                  pl.BlockSpec((1, hdim), lambda i: (0, 0)),
                  pl.BlockSpec((hdim, EMB), lambda i: (0, 0))],
        out_specs=pl.BlockSpec((N, EMB), lambda i: (0, 0)),
        out_shape=jax.ShapeDtypeStruct((N, EMB), jnp.float32),
        scratch_shapes=[pltpu.VMEM((N, hdim), jnp.float32)],
    )(a_flat, xw1, b.reshape(1, hdim), w2)


def _dao_pos_body(h_ref, nz_ref, dw1h_ref, dw1n_ref, db1_ref, a1_ref,
                  dw2_ref, db2_ref, a2_ref, swc_ref, aug_ref, cat_ref,
                  pos_ref):
    t = (jnp.dot(h_ref[...], dw1h_ref[...], preferred_element_type=jnp.float32)
         + jnp.dot(nz_ref[...], dw1n_ref[...],
                   preferred_element_type=jnp.float32)
         + db1_ref[...])
    t = jnp.maximum(t, 0.0) + a1_ref[...] * jnp.minimum(t, 0.0)
    u = jnp.dot(t, dw2_ref[...], preferred_element_type=jnp.float32) + db2_ref[...]
    aug = jnp.maximum(u, 0.0) + a2_ref[...] * jnp.minimum(u, 0.0)
    aug_ref[...] = aug
    cat_ref[0:N, :] = aug
    cat_ref[N:2 * N, :] = aug
    d = jnp.abs(aug - h_ref[...])
    logit = jnp.dot(d, swc_ref[...], preferred_element_type=jnp.float32)
    ce = jnp.log1p(jnp.exp(-jnp.abs(logit))) + jnp.maximum(-logit, 0.0)
    pos_ref[0, 0] = jnp.sum(ce)
    cs_h = jnp.sum(h_ref[...], axis=0, keepdims=True)
    cs_a = jnp.sum(aug, axis=0, keepdims=True)
    pos_ref[0, 1] = jnp.sum(cs_h * cs_h)
    pos_ref[0, 2] = jnp.sum(cs_a * cs_h)


def _dao_pos(h, noise, dw1, db1, a1, dw2, db2, a2, sw):
    return pl.pallas_call(
        _dao_pos_body,
        out_specs=(pl.BlockSpec(memory_space=pltpu.VMEM),
                   pl.BlockSpec(memory_space=pltpu.VMEM),
                   pl.BlockSpec(memory_space=pltpu.SMEM)),
        out_shape=(jax.ShapeDtypeStruct((N, EMB), jnp.float32),
                   jax.ShapeDtypeStruct((AUG * N, EMB), jnp.float32),
                   jax.ShapeDtypeStruct((1, 3), jnp.float32)),
    )(h, noise, dw1[:EMB], dw1[EMB:], db1.reshape(1, HID), a1.reshape(1, HID),
      dw2, db2.reshape(1, EMB), a2.reshape(1, EMB), sw.reshape(EMB, 1))


def _labmm_body(lab_ref, h_ref, aug_ref, hi_ref, out_ref, acc_ref):
    i = pl.program_id(0)

    @pl.when(i == 0)
    def _():
        acc_ref[0] = 0.0
        acc_ref[1] = 0.0

    lab2 = lab_ref[...].reshape(_GCN_BLK, N)
    m = jnp.dot(lab2, h_ref[...], preferred_element_type=jnp.float32)
    acc_ref[0] += jnp.sum(hi_ref[...] * m)
    acc_ref[1] += jnp.sum(aug_ref[...] * m)

    @pl.when(i == N // _GCN_BLK - 1)
    def _():
        out_ref[0, 0] = acc_ref[0]
        out_ref[0, 1] = acc_ref[1]


def _labmm_sums(lab_flat, h, aug_h):
    return pl.pallas_call(
        _labmm_body,
        grid=(N // _GCN_BLK,),
        in_specs=[pl.BlockSpec((_GCN_BLK * N,), lambda i: (i,)),
                  pl.BlockSpec((N, EMB), lambda i: (0, 0)),
                  pl.BlockSpec((_GCN_BLK, EMB), lambda i: (i, 0)),
                  pl.BlockSpec((_GCN_BLK, EMB), lambda i: (i, 0))],
        out_specs=pl.BlockSpec(memory_space=pltpu.SMEM),
        out_shape=jax.ShapeDtypeStruct((1, 2), jnp.float32),
        scratch_shapes=[pltpu.SMEM((2,), jnp.float32)],
    )(lab_flat, h, aug_h, h)


_CE_BI = 256


def _ce_body(hi_ref, hj_ref, ai_ref, s_ref, acc_ref):
    i = pl.program_id(0)

    @pl.when(i == 0)
    def _():
        acc_ref[0] = 0.0
        acc_ref[1] = 0.0

    dn = (((1,), (1,)), ((), ()))
    rec1 = lax.dot_general(hi_ref[...], hj_ref[...], dn,
                           preferred_element_type=jnp.float32)
    rec2 = lax.dot_general(ai_ref[...], hj_ref[...], dn,
                           preferred_element_type=jnp.float32)

    sp1 = jnp.log1p(jnp.exp(-rec1))
    sp2 = jnp.log1p(jnp.exp(-jnp.abs(rec2))) + jnp.maximum(-rec2, 0.0)

    acc_ref[0] += jnp.sum(sp1)
    acc_ref[1] += jnp.sum(sp2)

    @pl.when(i == N // _CE_BI - 1)
    def _():
        s_ref[0, 0] = acc_ref[0]
        s_ref[0, 1] = acc_ref[1]


def _ce_sums(h, aug_h):
    return pl.pallas_call(
        _ce_body,
        grid=(N // _CE_BI,),
        in_specs=[pl.BlockSpec((_CE_BI, EMB), lambda i: (i, 0)),
                  pl.BlockSpec((N, EMB), lambda i: (0, 0)),
                  pl.BlockSpec((_CE_BI, EMB), lambda i: (i, 0))],
        out_specs=pl.BlockSpec(memory_space=pltpu.SMEM),
        out_shape=jax.ShapeDtypeStruct((1, 2), jnp.float32),
        scratch_shapes=[pltpu.SMEM((2,), jnp.float32)],
    )(h, h, aug_h)


def _sia_body(negb_ref, cat_ref, swc_ref, out_ref, acc_ref):
    j = pl.program_id(0)

    @pl.when(j == 0)
    def _():
        acc_ref[0] = 0.0

    d = jnp.abs(cat_ref[...] - negb_ref[...])
    logit = jnp.dot(d, swc_ref[...], preferred_element_type=jnp.float32)
    ce = logit + jnp.log1p(jnp.exp(-jnp.abs(logit))) + jnp.maximum(-logit, 0.0)
    acc_ref[0] += jnp.sum(ce)

    @pl.when(j == NEG - 1)
    def _():
        out_ref[0, 0] = acc_ref[0]


def _sia_neg_sum(neg_h, aug_cat, sw):
    return pl.pallas_call(
        _sia_body,
        grid=(NEG,),
        in_specs=[pl.BlockSpec((AUG * N, EMB), lambda j: (j, 0)),
                  pl.BlockSpec((AUG * N, EMB), lambda j: (0, 0)),
                  pl.BlockSpec((EMB, 1), lambda j: (0, 0))],
        out_specs=pl.BlockSpec(memory_space=pltpu.SMEM),
        out_shape=jax.ShapeDtypeStruct((1, 1), jnp.float32),
        scratch_shapes=[pltpu.SMEM((1,), jnp.float32)],
    )(neg_h, aug_cat, sw.reshape(EMB, 1))


def kernel(x, adj_weight, aug_noise, W1, b1, W2, b2, dW1, db1, a1, dW2, db2,
           a2, siamese_w, edge_index, adj_orig_index, negative_index):
    xw1 = _matmul(x, W1)
    a_flat = _build_adj(edge_index[0], edge_index[1], adj_weight)
    xw2 = _gcn1_fused(a_flat, xw1, b1, W2)
    lc_flat = _build_lab(adj_orig_index[0], adj_orig_index[1])
    h = _gcn_layer_pre(a_flat, xw2, b2)

    aug_h, aug_cat, dao_sums = _dao_pos(h, aug_noise, dW1, db1, a1, dW2, db2,
                                        a2, siamese_w)
    neg_h = _neg_gather(h, negative_index.T.reshape(-1))
    sp_sums = _ce_sums(h, aug_h)
    lab_sums = _labmm_sums(lc_flat, h, aug_h)
    neg_sum = _sia_neg_sum(neg_h, aug_cat, siamese_w)

    pos_sum = dao_sums[0, 0]
    s1 = dao_sums[0, 1] - lab_sums[0, 0] + sp_sums[0, 0]
    s2 = dao_sums[0, 2] - lab_sums[0, 1] + sp_sums[0, 1]

    nn = float(N * N)
    gae_l = NORM * s1 / nn
    aug_gae_l = (NORM * s2 / nn) * AUG_GAE_W
    n_sia = float(AUG * N + AUG * N * NEG)
    sia_l = ((AUG * pos_sum + neg_sum[0, 0]) / n_sia) * SIA_LOSS_W
    total = gae_l + aug_gae_l + sia_l
    return total, gae_l, aug_gae_l, sia_l, h, aug_cat


# speedup vs baseline: 1.4107x; 1.0024x over previous
import functools

import jax
import jax.numpy as jnp
from jax import lax
from jax.experimental import pallas as pl
from jax.experimental.pallas import tpu as pltpu
from jax.experimental.pallas import tpu_sc as plsc

N = 2048
D = 256
E = 65536
HID = 256
EMB = 128
NOISE_DIM = 16
AUG = 2
NEG = 10
NORM = 0.1
AUG_GAE_W = 1e-05
SIA_LOSS_W = 1e-05

_NSC = 2
_NTILE = 16
_BLK_ROWS = N // 4
_SP_WORDS = _BLK_ROWS * N
_TRASH = N
_EPT = E // _NTILE
_CHUNK = 128
_NCHUNK = _EPT // _CHUNK
_ZCH = 8192
_ZSTRIDE = (_SP_WORDS + _TRASH) // _NTILE
_DSTRIDE = _SP_WORDS // _NTILE

_sc_mesh = functools.partial(
    plsc.VectorSubcoreMesh, core_axis_name="c", subcore_axis_name="s")


_BUILD_SCRATCH = [
    pltpu.VMEM_SHARED((_SP_WORDS + _TRASH,), jnp.float32),
    pltpu.VMEM((_EPT,), jnp.int32),
    pltpu.VMEM((_EPT,), jnp.int32),
    pltpu.VMEM((_EPT,), jnp.float32),
    pltpu.VMEM((_NCHUNK, _CHUNK), jnp.int32),
    pltpu.VMEM((_NCHUNK, _CHUNK), jnp.float32),
    pltpu.VMEM((_ZCH,), jnp.float32),
    pltpu.SemaphoreType.DMA,
]


def _scatter_build_body(use_w, erow, ecol, ew, out,
                        spm, rbuf, cbuf, wbuf, idxbuf, valbuf, zbuf, sem):
    c = lax.axis_index("c")
    s = lax.axis_index("s")

    zero16 = jnp.zeros((16,), jnp.float32)

    def _zfill(i, carry):
        zbuf[pl.ds(i * 16, 16)] = zero16
        return carry
    lax.fori_loop(0, _ZCH // 16, _zfill, 0)

    for p in range(2):
        base = (c * 2 + p) * _BLK_ROWS

        for k in range(_ZSTRIDE // _ZCH):
            pltpu.sync_copy(zbuf, spm.at[pl.ds(s * _ZSTRIDE + k * _ZCH, _ZCH)])
        rem = _ZSTRIDE % _ZCH
        if rem:
            pltpu.sync_copy(zbuf.at[pl.ds(0, rem)],
                            spm.at[pl.ds(s * _ZSTRIDE + _ZSTRIDE - rem, rem)])
        plsc.subcore_barrier()

        eb = s * _EPT
        pltpu.sync_copy(erow.at[pl.ds(eb, _EPT)], rbuf)
        pltpu.sync_copy(ecol.at[pl.ds(eb, _EPT)], cbuf)
        if use_w:
            pltpu.sync_copy(ew.at[pl.ds(eb, _EPT)], wbuf)

        copies = []
        for j in range(_NCHUNK):
            def _grp(g, carry, _j=j):
                o = _j * _CHUNK + g * 16
                r16 = rbuf[pl.ds(o, 16)]
                c16 = cbuf[pl.ds(o, 16)]
                inb = (r16 >= base) & (r16 < base + _BLK_ROWS)
                idx16 = jnp.where(inb, (r16 - base) * N + c16,
                                  _SP_WORDS + c16)
                if use_w:
                    v16 = jnp.where(inb, wbuf[pl.ds(o, 16)], 0.0)
                else:
                    v16 = jnp.where(inb, 1.0, 0.0)
                idxbuf[_j, pl.ds(g * 16, 16)] = idx16
                valbuf[_j, pl.ds(g * 16, 16)] = v16
                return carry
            lax.fori_loop(0, _CHUNK // 16, _grp, 0)
            copies.append(
                pltpu.async_copy(valbuf.at[j], spm.at[idxbuf.at[j]], sem,
                                 add=use_w))
        for cp in copies:
            cp.wait()
        plsc.subcore_barrier()

        pltpu.sync_copy(spm.at[pl.ds(s * _DSTRIDE, _DSTRIDE)],
                        out.at[pl.ds(base * N + s * _DSTRIDE, _DSTRIDE)])
        plsc.subcore_barrier()


@functools.partial(
    pl.kernel,
    out_type=jax.ShapeDtypeStruct((N * N,), jnp.float32),
    mesh=_sc_mesh(),
    scratch_types=_BUILD_SCRATCH,
)
def _build_adj(erow, ecol, ew, out, *scratch):
    _scatter_build_body(True, erow, ecol, ew, out, *scratch)


@functools.partial(
    pl.kernel,
    out_type=jax.ShapeDtypeStruct((N * N,), jnp.float32),
    mesh=_sc_mesh(),
    scratch_types=_BUILD_SCRATCH,
)
def _build_lab(erow, ecol, out, *scratch):
    _scatter_build_body(False, erow, ecol, None, out, *scratch)


_GB = AUG * N * NEG
_GW = _GB // (_NSC * _NTILE)
_GCH = 128


@functools.partial(
    pl.kernel,
    out_type=jax.ShapeDtypeStruct((_GB, EMB), jnp.float32),
    mesh=_sc_mesh(),
    scratch_types=[
        pltpu.VMEM((_GW,), jnp.int32),
        pltpu.VMEM((_GCH, EMB), jnp.float32),
        pltpu.VMEM((_GCH, EMB), jnp.float32),
        pltpu.SemaphoreType.DMA,
        pltpu.SemaphoreType.DMA,
        pltpu.SemaphoreType.DMA,
        pltpu.SemaphoreType.DMA,
    ],
)
def _neg_gather(h_hbm, idx_hbm, out_hbm, idx_v, rows_a, rows_b,
                gsem_a, gsem_b, wsem_a, wsem_b):
    c = lax.axis_index("c")
    s = lax.axis_index("s")
    wid = s * _NSC + c
    base = wid * _GW
    pltpu.sync_copy(idx_hbm.at[pl.ds(base, _GW)], idx_v)

    bufs = ((rows_a, gsem_a, wsem_a), (rows_b, gsem_b, wsem_b))
    nch = _GW // _GCH
    gathers = [None, None]
    writes = [None, None]

    def _gather(g):
        buf, gsem, _ = bufs[g % 2]
        return pltpu.async_copy(h_hbm.at[idx_v.at[pl.ds(g * _GCH, _GCH)]],
                                buf, gsem)

    gathers[0] = _gather(0)
    for g in range(nch):
        b = g % 2
        nb = (g + 1) % 2
        gathers[b].wait()
        if g + 1 < nch:
            if writes[nb] is not None:
                writes[nb].wait()
            gathers[nb] = _gather(g + 1)
        buf, _, wsem = bufs[b]
        writes[b] = pltpu.async_copy(
            buf, out_hbm.at[pl.ds(base + g * _GCH, _GCH)], wsem)
    writes[0].wait()
    writes[1].wait()


def _matmul_body(x_ref, w_ref, out_ref):
    out_ref[...] = jnp.dot(x_ref[...], w_ref[...],
                           preferred_element_type=jnp.float32)


def _matmul(x, w):
    return pl.pallas_call(
        _matmul_body,
        out_shape=jax.ShapeDtypeStruct((x.shape[0], w.shape[1]), jnp.float32),
    )(x, w)


_GCN_BLK = 256


def _gcn_xw_body(a_ref, xw_ref, b_ref, out_ref):
    a2 = a_ref[...].reshape(_GCN_BLK, N)
    agg = jnp.dot(a2, xw_ref[...], preferred_element_type=jnp.float32)
    out_ref[...] = jnp.maximum(agg + b_ref[...], 0.0)


def _gcn_layer_pre(a_flat, xw, b):
    hdim = xw.shape[1]
    return pl.pallas_call(
        _gcn_xw_body,
        grid=(N // _GCN_BLK,),
        in_specs=[pl.BlockSpec((_GCN_BLK * N,), lambda i: (i,)),
                  pl.BlockSpec((N, hdim), lambda i: (0, 0)),
                  pl.BlockSpec((1, hdim), lambda i: (0, 0))],
        out_specs=pl.BlockSpec((_GCN_BLK, hdim), lambda i: (i, 0)),
        out_shape=jax.ShapeDtypeStruct((N, hdim), jnp.float32),
    )(a_flat, xw, b.reshape(1, hdim))


def _gcn1_body(a_ref, xw1_ref, b_ref, w2_ref, xw2_ref, h1s_ref):
    i = pl.program_id(0)
    a2 = a_ref[...].reshape(_GCN_BLK, N)
    agg = jnp.dot(a2, xw1_ref[...], preferred_element_type=jnp.float32)
    h1s_ref[pl.ds(i * _GCN_BLK, _GCN_BLK), :] = jnp.maximum(
        agg + b_ref[...], 0.0)

    @pl.when(i == N // _GCN_BLK - 1)
    def _():
        xw2_ref[...] = jnp.dot(h1s_ref[...], w2_ref[...],
                               preferred_element_type=jnp.float32)


def _gcn1_fused(a_flat, xw1, b, w2):
    hdim = xw1.shape[1]
    return pl.pallas_call(
        _gcn1_body,
        grid=(N // _GCN_BLK,),
        in_specs=[pl.BlockSpec((_GCN_BLK * N,), lambda i: (i,)),
                  pl.BlockSpec((N, hdim), lambda i: (0, 0)),
                  pl.BlockSpec((1, hdim), lambda i: (0, 0)),
                  pl.BlockSpec((hdim, EMB), lambda i: (0, 0))],
        out_specs=pl.BlockSpec((N, EMB), lambda i: (0, 0)),
        out_shape=jax.ShapeDtypeStruct((N, EMB), jnp.float32),
        scratch_shapes=[pltpu.VMEM((N, hdim), jnp.float32)],
    )(a_flat, xw1, b.reshape(1, hdim), w2)


def _dao_pos_body(h_ref, nz_ref, dw1h_ref, dw1n_ref, db1_ref, a1_ref,
                  dw2_ref, db2_ref, a2_ref, swc_ref, aug_ref, cat_ref,
                  pos_ref):
    t = (jnp.dot(h_ref[...], dw1h_ref[...], preferred_element_type=jnp.float32)
         + jnp.dot(nz_ref[...], dw1n_ref[...],
                   preferred_element_type=jnp.float32)
         + db1_ref[...])
    t = jnp.maximum(t, 0.0) + a1_ref[...] * jnp.minimum(t, 0.0)
    u = jnp.dot(t, dw2_ref[...], preferred_element_type=jnp.float32) + db2_ref[...]
    aug = jnp.maximum(u, 0.0) + a2_ref[...] * jnp.minimum(u, 0.0)
    aug_ref[...] = aug
    cat_ref[0:N, :] = aug
    cat_ref[N:2 * N, :] = aug
    d = jnp.abs(aug - h_ref[...])
    logit = jnp.dot(d, swc_ref[...], preferred_element_type=jnp.float32)
    ce = jnp.log1p(jnp.exp(-jnp.abs(logit))) + jnp.maximum(-logit, 0.0)
    pos_ref[0, 0] = jnp.sum(ce)
    cs_h = jnp.sum(h_ref[...], axis=0, keepdims=True)
    cs_a = jnp.sum(aug, axis=0, keepdims=True)
    pos_ref[0, 1] = jnp.sum(cs_h * cs_h)
    pos_ref[0, 2] = jnp.sum(cs_a * cs_h)


def _dao_pos(h, noise, dw1, db1, a1, dw2, db2, a2, sw):
    return pl.pallas_call(
        _dao_pos_body,
        out_specs=(pl.BlockSpec(memory_space=pltpu.VMEM),
                   pl.BlockSpec(memory_space=pltpu.VMEM),
                   pl.BlockSpec(memory_space=pltpu.SMEM)),
        out_shape=(jax.ShapeDtypeStruct((N, EMB), jnp.float32),
                   jax.ShapeDtypeStruct((AUG * N, EMB), jnp.float32),
                   jax.ShapeDtypeStruct((1, 3), jnp.float32)),
    )(h, noise, dw1[:EMB], dw1[EMB:], db1.reshape(1, HID), a1.reshape(1, HID),
      dw2, db2.reshape(1, EMB), a2.reshape(1, EMB), sw.reshape(EMB, 1))


def _labmm_body(lab_ref, h_ref, aug_ref, hi_ref, out_ref, acc_ref):
    i = pl.program_id(0)

    @pl.when(i == 0)
    def _():
        acc_ref[0] = 0.0
        acc_ref[1] = 0.0

    lab2 = lab_ref[...].reshape(_GCN_BLK, N)
    m = jnp.dot(lab2, h_ref[...], preferred_element_type=jnp.float32)
    acc_ref[0] += jnp.sum(hi_ref[...] * m)
    acc_ref[1] += jnp.sum(aug_ref[...] * m)

    @pl.when(i == N // _GCN_BLK - 1)
    def _():
        out_ref[0, 0] = acc_ref[0]
        out_ref[0, 1] = acc_ref[1]


def _labmm_sums(lab_flat, h, aug_h):
    return pl.pallas_call(
        _labmm_body,
        grid=(N // _GCN_BLK,),
        in_specs=[pl.BlockSpec((_GCN_BLK * N,), lambda i: (i,)),
                  pl.BlockSpec((N, EMB), lambda i: (0, 0)),
                  pl.BlockSpec((_GCN_BLK, EMB), lambda i: (i, 0)),
                  pl.BlockSpec((_GCN_BLK, EMB), lambda i: (i, 0))],
        out_specs=pl.BlockSpec(memory_space=pltpu.SMEM),
        out_shape=jax.ShapeDtypeStruct((1, 2), jnp.float32),
        scratch_shapes=[pltpu.SMEM((2,), jnp.float32)],
    )(lab_flat, h, aug_h, h)


_CE_BI = 512


def _ce_body(hi_ref, hj_ref, ai_ref, s_ref, acc_ref):
    i = pl.program_id(0)

    @pl.when(i == 0)
    def _():
        acc_ref[0] = 0.0
        acc_ref[1] = 0.0

    dn = (((1,), (1,)), ((), ()))
    rec1 = lax.dot_general(hi_ref[...], hj_ref[...], dn,
                           preferred_element_type=jnp.float32)
    rec2 = lax.dot_general(ai_ref[...], hj_ref[...], dn,
                           preferred_element_type=jnp.float32)

    sp1 = jnp.log1p(jnp.exp(-rec1))
    sp2 = jnp.log1p(jnp.exp(-jnp.abs(rec2))) + jnp.maximum(-rec2, 0.0)

    acc_ref[0] += jnp.sum(sp1)
    acc_ref[1] += jnp.sum(sp2)

    @pl.when(i == N // _CE_BI - 1)
    def _():
        s_ref[0, 0] = acc_ref[0]
        s_ref[0, 1] = acc_ref[1]


def _ce_sums(h, aug_h):
    return pl.pallas_call(
        _ce_body,
        grid=(N // _CE_BI,),
        in_specs=[pl.BlockSpec((_CE_BI, EMB), lambda i: (i, 0)),
                  pl.BlockSpec((N, EMB), lambda i: (0, 0)),
                  pl.BlockSpec((_CE_BI, EMB), lambda i: (i, 0))],
        out_specs=pl.BlockSpec(memory_space=pltpu.SMEM),
        out_shape=jax.ShapeDtypeStruct((1, 2), jnp.float32),
        scratch_shapes=[pltpu.SMEM((2,), jnp.float32)],
    )(h, h, aug_h)


def _sia_body(negb_ref, cat_ref, swc_ref, out_ref, acc_ref):
    j = pl.program_id(0)

    @pl.when(j == 0)
    def _():
        acc_ref[0] = 0.0

    d = jnp.abs(cat_ref[...] - negb_ref[...])
    logit = jnp.dot(d, swc_ref[...], preferred_element_type=jnp.float32)
    ce = logit + jnp.log1p(jnp.exp(-jnp.abs(logit))) + jnp.maximum(-logit, 0.0)
    acc_ref[0] += jnp.sum(ce)

    @pl.when(j == NEG - 1)
    def _():
        out_ref[0, 0] = acc_ref[0]


def _sia_neg_sum(neg_h, aug_cat, sw):
    return pl.pallas_call(
        _sia_body,
        grid=(NEG,),
        in_specs=[pl.BlockSpec((AUG * N, EMB), lambda j: (j, 0)),
                  pl.BlockSpec((AUG * N, EMB), lambda j: (0, 0)),
                  pl.BlockSpec((EMB, 1), lambda j: (0, 0))],
        out_specs=pl.BlockSpec(memory_space=pltpu.SMEM),
        out_shape=jax.ShapeDtypeStruct((1, 1), jnp.float32),
        scratch_shapes=[pltpu.SMEM((1,), jnp.float32)],
    )(neg_h, aug_cat, sw.reshape(EMB, 1))


def kernel(x, adj_weight, aug_noise, W1, b1, W2, b2, dW1, db1, a1, dW2, db2,
           a2, siamese_w, edge_index, adj_orig_index, negative_index):
    xw1 = _matmul(x, W1)
    a_flat = _build_adj(edge_index[0], edge_index[1], adj_weight)
    xw2 = _gcn1_fused(a_flat, xw1, b1, W2)
    lc_flat = _build_lab(adj_orig_index[0], adj_orig_index[1])
    h = _gcn_layer_pre(a_flat, xw2, b2)

    aug_h, aug_cat, dao_sums = _dao_pos(h, aug_noise, dW1, db1, a1, dW2, db2,
                                        a2, siamese_w)
    neg_h = _neg_gather(h, negative_index.T.reshape(-1))
    sp_sums = _ce_sums(h, aug_h)
    lab_sums = _labmm_sums(lc_flat, h, aug_h)
    neg_sum = _sia_neg_sum(neg_h, aug_cat, siamese_w)

    pos_sum = dao_sums[0, 0]
    s1 = dao_sums[0, 1] - lab_sums[0, 0] + sp_sums[0, 0]
    s2 = dao_sums[0, 2] - lab_sums[0, 1] + sp_sums[0, 1]

    nn = float(N * N)
    gae_l = NORM * s1 / nn
    aug_gae_l = (NORM * s2 / nn) * AUG_GAE_W
    n_sia = float(AUG * N + AUG * N * NEG)
    sia_l = ((AUG * pos_sum + neg_sum[0, 0]) / n_sia) * SIA_LOSS_W
    total = gae_l + aug_gae_l + sia_l
    return total, gae_l, aug_gae_l, sia_l, h, aug_cat
